# initial kernel scaffold (unmeasured)
import jax
import jax.numpy as jnp
from jax import lax
from jax.experimental import pallas as pl
from jax.experimental.pallas import tpu as pltpu

N_DEV = 4
E_LOC = 8
N_EXP = 32


def kernel(x, router_W, route_idx, expert_W):
    n_tok, d = x.shape
    e_loc, _, h = expert_W.shape

    def body(x_ref, rw_ref, ridx_ref, ew_ref, out_ref,
             xc, gc, rs, w_bf,
             x_send, x_recv, g_send, g_recv, rs_send, rs_recv):
        my = lax.axis_index("i")
        left = (my + N_DEV - 1) % N_DEV
        right = (my + 1) % N_DEV

        barrier = pltpu.get_barrier_semaphore()
        for nbr in (left, right):
            pl.semaphore_signal(barrier, inc=1, device_id=(nbr,),
                                device_id_type=pl.DeviceIdType.MESH)
        pl.semaphore_wait(barrier, 2)

        xf = x_ref[:, :]
        scores = jnp.dot(xf, rw_ref[:, :], preferred_element_type=jnp.float32)
        p = jnp.exp(scores - jnp.max(scores, axis=1, keepdims=True))
        p = p / jnp.sum(p, axis=1, keepdims=True)
        eids = lax.broadcasted_iota(jnp.int32, (n_tok, N_EXP), 1)
        picked = (eids == ridx_ref[:, 0:1]) | (eids == ridx_ref[:, 1:2])
        g = jnp.where(picked, p, 0.0)
        g = g / jnp.sum(g, axis=1, keepdims=True)
        gc[0, :, :] = g
        xc[0, :, :] = xf.astype(jnp.bfloat16)
        w_bf[...] = ew_ref[...].astype(jnp.bfloat16)

        for hh in range(1, N_DEV):
            xr = pltpu.make_async_remote_copy(
                src_ref=xc.at[hh - 1], dst_ref=xc.at[hh],
                send_sem=x_send.at[hh - 1], recv_sem=x_recv.at[hh - 1],
                device_id=(right,), device_id_type=pl.DeviceIdType.MESH)
            gr = pltpu.make_async_remote_copy(
                src_ref=gc.at[hh - 1], dst_ref=gc.at[hh],
                send_sem=g_send.at[hh - 1], recv_sem=g_recv.at[hh - 1],
                device_id=(right,), device_id_type=pl.DeviceIdType.MESH)
            xr.start()
            gr.start()
            xr.wait()
            gr.wait()

        def partial(slot):
            xch = xc[slot, :, :]
            gall = gc[slot, :, :]
            acc = None
            for e in range(E_LOC):
                ge = jnp.sum(jnp.where(eids == my * E_LOC + e, gall, 0.0),
                             axis=1, keepdims=True)
                xe = xch * ge.astype(jnp.bfloat16)
                y = jnp.dot(xe, w_bf[e, :, :],
                            preferred_element_type=jnp.float32)
                acc = y if acc is None else acc + y
            return acc

        rs[0, :, :] = partial(1)
        for s in range(N_DEV - 1):
            rr = pltpu.make_async_remote_copy(
                src_ref=rs.at[s], dst_ref=rs.at[s + 1],
                send_sem=rs_send.at[s], recv_sem=rs_recv.at[s],
                device_id=(right,), device_id_type=pl.DeviceIdType.MESH)
            rr.start()
            rr.wait()
            if s < N_DEV - 2:
                rs[s + 1, :, :] += partial(s + 2)
        out_ref[:, :] = rs[N_DEV - 1, :, :] + partial(0)

    return pl.pallas_call(
        body,
        out_shape=jax.ShapeDtypeStruct((n_tok, h), jnp.float32),
        in_specs=[pl.BlockSpec(memory_space=pltpu.VMEM)] * 4,
        out_specs=pl.BlockSpec(memory_space=pltpu.VMEM),
        scratch_shapes=[
            pltpu.VMEM((N_DEV, n_tok, d), jnp.bfloat16),
            pltpu.VMEM((N_DEV, n_tok, N_EXP), jnp.float32),
            pltpu.VMEM((N_DEV, n_tok, h), jnp.float32),
            pltpu.VMEM((e_loc, d, h), jnp.bfloat16),
            pltpu.SemaphoreType.DMA((N_DEV - 1,)),
            pltpu.SemaphoreType.DMA((N_DEV - 1,)),
            pltpu.SemaphoreType.DMA((N_DEV - 1,)),
            pltpu.SemaphoreType.DMA((N_DEV - 1,)),
            pltpu.SemaphoreType.DMA((N_DEV - 1,)),
            pltpu.SemaphoreType.DMA((N_DEV - 1,)),
        ],
        compiler_params=pltpu.CompilerParams(collective_id=0),
    )(x, router_W, route_idx, expert_W)


# baseline (device time: 323675 ns/iter reference)
import jax
import jax.numpy as jnp
from jax import lax
from jax.experimental import pallas as pl
from jax.experimental.pallas import tpu as pltpu

N_DEV = 4
E_LOC = 8
N_EXP = 32


def kernel(x, router_W, route_idx, expert_W):
    n_tok, d = x.shape
    e_loc, _, h = expert_W.shape

    def body(x_ref, rw_ref, ridx_ref, ew_ref, out_ref,
             xc, gc, rs,
             x_send, x_recv, g_send, g_recv, rs_send, rs_recv):
        my = lax.axis_index("i")
        left = (my + N_DEV - 1) % N_DEV
        right = (my + 1) % N_DEV

        barrier = pltpu.get_barrier_semaphore()
        for nbr in (left, right):
            pl.semaphore_signal(barrier, inc=1, device_id=(nbr,),
                                device_id_type=pl.DeviceIdType.MESH)
        pl.semaphore_wait(barrier, 2)

        scores = jnp.dot(x_ref[:, :], rw_ref[:, :],
                         preferred_element_type=jnp.float32)
        p = jnp.exp(scores - jnp.max(scores, axis=1, keepdims=True))
        p = p / jnp.sum(p, axis=1, keepdims=True)
        eids = lax.broadcasted_iota(jnp.int32, (n_tok, N_EXP), 1)
        picked = (eids == ridx_ref[:, 0:1]) | (eids == ridx_ref[:, 1:2])
        g = jnp.where(picked, p, 0.0)
        g = g / jnp.sum(g, axis=1, keepdims=True)
        gc[0, :, :] = g.astype(jnp.bfloat16)
        xc[0, :, :] = x_ref[:, :]

        for hh in range(1, N_DEV):
            xr = pltpu.make_async_remote_copy(
                src_ref=xc.at[hh - 1], dst_ref=xc.at[hh],
                send_sem=x_send.at[hh - 1], recv_sem=x_recv.at[hh - 1],
                device_id=(right,), device_id_type=pl.DeviceIdType.MESH)
            gr = pltpu.make_async_remote_copy(
                src_ref=gc.at[hh - 1], dst_ref=gc.at[hh],
                send_sem=g_send.at[hh - 1], recv_sem=g_recv.at[hh - 1],
                device_id=(right,), device_id_type=pl.DeviceIdType.MESH)
            xr.start()
            gr.start()
            xr.wait()
            gr.wait()

        def accum_partial(ag_slot, write):
            xch = xc[ag_slot, :, :]
            gall = gc[ag_slot, :, :]
            for e in range(E_LOC):
                ge = jnp.sum(jnp.where(eids == my * E_LOC + e, gall, 0),
                             axis=1, keepdims=True)
                y = jnp.dot(xch * ge, ew_ref[e, :, :],
                            preferred_element_type=jnp.float32)
                write(e, y)

        def into_rs(slot, init):
            def write(e, y):
                if init and e == 0:
                    rs[slot, :, :] = y.astype(jnp.bfloat16)
                else:
                    rs[slot, :, :] = rs[slot, :, :] + y.astype(jnp.bfloat16)
            return write

        accum_partial(1, into_rs(0, init=True))
        for s in range(N_DEV - 1):
            sb, rb = s % 3, (s + 1) % 3
            rr = pltpu.make_async_remote_copy(
                src_ref=rs.at[sb], dst_ref=rs.at[rb],
                send_sem=rs_send.at[s], recv_sem=rs_recv.at[s],
                device_id=(right,), device_id_type=pl.DeviceIdType.MESH)
            rr.start()
            rr.wait()
            if s < N_DEV - 2:
                accum_partial(s + 2, into_rs(rb, init=False))

        out_ref[:, :] = rs[0, :, :].astype(jnp.float32)

        def into_out(e, y):
            out_ref[:, :] = out_ref[:, :] + y
        accum_partial(0, into_out)

    out = pl.pallas_call(
        body,
        out_shape=jax.ShapeDtypeStruct((n_tok, h), jnp.float32),
        in_specs=[pl.BlockSpec(memory_space=pltpu.VMEM)] * 4,
        out_specs=pl.BlockSpec(memory_space=pltpu.VMEM),
        scratch_shapes=[
            pltpu.VMEM((N_DEV, n_tok, d), jnp.bfloat16),
            pltpu.VMEM((N_DEV, n_tok, N_EXP), jnp.bfloat16),
            pltpu.VMEM((3, n_tok, h), jnp.bfloat16),
            pltpu.SemaphoreType.DMA((N_DEV - 1,)),
            pltpu.SemaphoreType.DMA((N_DEV - 1,)),
            pltpu.SemaphoreType.DMA((N_DEV - 1,)),
            pltpu.SemaphoreType.DMA((N_DEV - 1,)),
            pltpu.SemaphoreType.DMA((N_DEV - 1,)),
            pltpu.SemaphoreType.DMA((N_DEV - 1,)),
        ],
        compiler_params=pltpu.CompilerParams(collective_id=0),
    )(
        x.astype(jnp.bfloat16),
        router_W.astype(jnp.bfloat16),
        route_idx,
        expert_W.astype(jnp.bfloat16),
    )
    return out


# device time: 247717 ns/iter; 1.3066x vs baseline; 1.3066x over previous
import jax
import jax.numpy as jnp
from jax import lax
from jax.experimental import pallas as pl
from jax.experimental.pallas import tpu as pltpu

N_DEV = 4
E_LOC = 8
N_EXP = 32


def kernel(x, router_W, route_idx, expert_W):
    n_tok, d = x.shape
    e_loc, _, h = expert_W.shape

    def body(x_ref, rw_ref, ridx_ref, ew_ref, out_ref,
             xc, gc, rs, pt,
             x_send, x_recv, g_send, g_recv, rs_send, rs_recv):
        my = lax.axis_index("i")
        left = (my + N_DEV - 1) % N_DEV
        right = (my + 1) % N_DEV

        barrier = pltpu.get_barrier_semaphore()
        for nbr in (left, right):
            pl.semaphore_signal(barrier, inc=1, device_id=(nbr,),
                                device_id_type=pl.DeviceIdType.MESH)
        pl.semaphore_wait(barrier, 2)

        scores = jnp.dot(x_ref[:, :], rw_ref[:, :],
                         preferred_element_type=jnp.float32)
        p = jnp.exp(scores - jnp.max(scores, axis=1, keepdims=True))
        p = p / jnp.sum(p, axis=1, keepdims=True)
        eids = lax.broadcasted_iota(jnp.int32, (n_tok, N_EXP), 1)
        picked = (eids == ridx_ref[:, 0:1]) | (eids == ridx_ref[:, 1:2])
        g = jnp.where(picked, p, 0.0)
        g = g / jnp.sum(g, axis=1, keepdims=True)
        gc[0, :, :] = g.astype(jnp.bfloat16)
        xc[0, :, :] = x_ref[:, :]

        def start_hop(hh):
            xr = pltpu.make_async_remote_copy(
                src_ref=xc.at[hh - 1], dst_ref=xc.at[hh],
                send_sem=x_send.at[hh - 1], recv_sem=x_recv.at[hh - 1],
                device_id=(right,), device_id_type=pl.DeviceIdType.MESH)
            gr = pltpu.make_async_remote_copy(
                src_ref=gc.at[hh - 1], dst_ref=gc.at[hh],
                send_sem=g_send.at[hh - 1], recv_sem=g_recv.at[hh - 1],
                device_id=(right,), device_id_type=pl.DeviceIdType.MESH)
            xr.start()
            gr.start()
            return xr, gr

        def rs_step(s):
            return pltpu.make_async_remote_copy(
                src_ref=rs.at[s % 3], dst_ref=rs.at[(s + 1) % 3],
                send_sem=rs_send.at[s], recv_sem=rs_recv.at[s],
                device_id=(right,), device_id_type=pl.DeviceIdType.MESH)

        TT = 512
        eids_t = lax.broadcasted_iota(jnp.int32, (TT, N_EXP), 1)

        def accum_partial(ag_slot, store):
            for t in range(0, n_tok, TT):
                xch = xc[ag_slot, pl.ds(t, TT), :]
                gall = gc[ag_slot, pl.ds(t, TT), :]
                acc = None
                for e in range(E_LOC):
                    ge = jnp.sum(jnp.where(eids_t == my * E_LOC + e, gall, 0),
                                 axis=1, keepdims=True)
                    y = jnp.dot(xch * ge, ew_ref[e, :, :],
                                preferred_element_type=jnp.float32)
                    acc = y if acc is None else acc + y
                store(t, acc)

        def into_buf(ref, slot):
            def store(t, acc):
                ref[slot, pl.ds(t, TT), :] = acc.astype(jnp.bfloat16)
            return store

        ag1 = start_hop(1)

        def into_out(t, acc):
            out_ref[pl.ds(t, TT), :] = acc
        accum_partial(0, into_out)

        ag1[0].wait()
        ag1[1].wait()
        ag2 = start_hop(2)
        accum_partial(1, into_buf(rs, 0))
        rr0 = rs_step(0)
        rr0.start()

        ag2[0].wait()
        ag2[1].wait()
        ag3 = start_hop(3)
        accum_partial(2, into_buf(pt, 0))
        ag3[0].wait()
        ag3[1].wait()

        rr0.wait_recv()
        rs[1, :, :] = rs[1, :, :] + pt[0, :, :]
        rr0.wait_send()
        rr1 = rs_step(1)
        rr1.start()
        accum_partial(3, into_buf(pt, 1))

        rr1.wait_recv()
        rs[2, :, :] = rs[2, :, :] + pt[1, :, :]
        rr1.wait_send()
        rr2 = rs_step(2)
        rr2.start()
        rr2.wait_recv()
        out_ref[:, :] = out_ref[:, :] + rs[0, :, :].astype(jnp.float32)
        rr2.wait_send()

    out = pl.pallas_call(
        body,
        out_shape=jax.ShapeDtypeStruct((n_tok, h), jnp.float32),
        in_specs=[pl.BlockSpec(memory_space=pltpu.VMEM)] * 4,
        out_specs=pl.BlockSpec(memory_space=pltpu.VMEM),
        scratch_shapes=[
            pltpu.VMEM((N_DEV, n_tok, d), jnp.bfloat16),
            pltpu.VMEM((N_DEV, n_tok, N_EXP), jnp.bfloat16),
            pltpu.VMEM((3, n_tok, h), jnp.bfloat16),
            pltpu.VMEM((2, n_tok, h), jnp.bfloat16),
            pltpu.SemaphoreType.DMA((N_DEV - 1,)),
            pltpu.SemaphoreType.DMA((N_DEV - 1,)),
            pltpu.SemaphoreType.DMA((N_DEV - 1,)),
            pltpu.SemaphoreType.DMA((N_DEV - 1,)),
            pltpu.SemaphoreType.DMA((N_DEV - 1,)),
            pltpu.SemaphoreType.DMA((N_DEV - 1,)),
        ],
        compiler_params=pltpu.CompilerParams(collective_id=0),
    )(
        x.astype(jnp.bfloat16),
        router_W.astype(jnp.bfloat16),
        route_idx,
        expert_W.astype(jnp.bfloat16),
    )
    return out


# device time: 177647 ns/iter; 1.8220x vs baseline; 1.3944x over previous
import jax
import jax.numpy as jnp
from jax import lax
from jax.experimental import pallas as pl
from jax.experimental.pallas import tpu as pltpu

N_DEV = 4
E_LOC = 8
N_EXP = 32


def kernel(x, router_W, route_idx, expert_W):
    n_tok, d = x.shape
    e_loc, _, h = expert_W.shape
    nh = n_tok // 2

    def body(x_ref, rw_ref, ridx_ref, ew_ref, out_ref,
             xc, gc, rsr, rsl,
             xr_send, xr_recv, gr_send, gr_recv,
             xl_send, xl_recv, gl_send, gl_recv,
             rsr_send, rsr_recv, rsl_send, rsl_recv):
        my = lax.axis_index("i")
        left = (my + N_DEV - 1) % N_DEV
        right = (my + 1) % N_DEV

        barrier = pltpu.get_barrier_semaphore()
        for nbr in (left, right):
            pl.semaphore_signal(barrier, inc=1, device_id=(nbr,),
                                device_id_type=pl.DeviceIdType.MESH)
        pl.semaphore_wait(barrier, 2)

        TT = 512
        eids_t = lax.broadcasted_iota(jnp.int32, (TT, N_EXP), 1)

        for t in range(0, n_tok, TT):
            sl = pl.ds(t, TT)
            scores = jnp.dot(x_ref[sl, :], rw_ref[:, :],
                             preferred_element_type=jnp.float32)
            p = jnp.exp(scores - jnp.max(scores, axis=1, keepdims=True))
            p = p / jnp.sum(p, axis=1, keepdims=True)
            picked = ((eids_t == ridx_ref[sl, 0:1])
                      | (eids_t == ridx_ref[sl, 1:2]))
            g = jnp.where(picked, p, 0.0)
            g = g / jnp.sum(g, axis=1, keepdims=True)
            gc[0, sl, :] = g.astype(jnp.bfloat16)

        def ag_hop(xsrc, xdst, gsrc, gdst, x_sems, g_sems, slot, dev):
            xr = pltpu.make_async_remote_copy(
                src_ref=xsrc, dst_ref=xdst,
                send_sem=x_sems[0].at[slot], recv_sem=x_sems[1].at[slot],
                device_id=(dev,), device_id_type=pl.DeviceIdType.MESH)
            gr = pltpu.make_async_remote_copy(
                src_ref=gc.at[gsrc], dst_ref=gc.at[gdst],
                send_sem=g_sems[0].at[slot], recv_sem=g_sems[1].at[slot],
                device_id=(dev,), device_id_type=pl.DeviceIdType.MESH)
            xr.start()
            gr.start()
            return xr, gr

        def rs_hop(ring, sems, s, dev):
            r = pltpu.make_async_remote_copy(
                src_ref=ring.at[s % 3], dst_ref=ring.at[(s + 1) % 3],
                send_sem=sems[0].at[s], recv_sem=sems[1].at[s],
                device_id=(dev,), device_id_type=pl.DeviceIdType.MESH)
            r.start()
            return r

        def accum_rows(xsrc, gslot, t0, t1, store):
            for t in range(t0, t1, TT):
                sl = pl.ds(t, TT)
                xch = xsrc(sl)
                gall = gc[gslot, sl, :]
                acc = None
                for e in range(E_LOC):
                    ge = jnp.sum(jnp.where(eids_t == my * E_LOC + e, gall, 0),
                                 axis=1, keepdims=True)
                    y = jnp.dot(xch * ge, ew_ref[e, :, :],
                                preferred_element_type=jnp.float32)
                    acc = y if acc is None else acc + y
                store(t, acc)

        own_x = lambda sl: x_ref[sl, :]
        slot_x = lambda k: (lambda sl: xc[k, sl, :])

        agr1 = ag_hop(x_ref, xc.at[0], 0, 1,
                      (xr_send, xr_recv), (gr_send, gr_recv), 0, right)
        agl1 = ag_hop(x_ref, xc.at[2], 0, 3,
                      (xl_send, xl_recv), (gl_send, gl_recv), 0, left)

        def st_out(t, acc):
            out_ref[pl.ds(t, TT), :] = acc
        accum_rows(own_x, 0, 0, n_tok, st_out)

        for r_ in agr1:
            r_.wait()
        agr2 = ag_hop(xc.at[0], xc.at[1], 1, 2,
                      (xr_send, xr_recv), (gr_send, gr_recv), 1, right)

        def st_rsr0(t, acc):
            rsr[0, pl.ds(t, TT), :] = acc.astype(jnp.bfloat16)
        accum_rows(slot_x(0), 1, 0, nh, st_rsr0)
        rr0 = rs_hop(rsr, (rsr_send, rsr_recv), 0, right)

        for r_ in agl1:
            r_.wait()

        def st_rsl0(t, acc):
            rsl[0, pl.ds(t - nh, TT), :] = acc.astype(jnp.bfloat16)
        accum_rows(slot_x(2), 3, nh, n_tok, st_rsl0)
        rl0 = rs_hop(rsl, (rsl_send, rsl_recv), 0, left)

        for r_ in agr2:
            r_.wait()

        def add_ring(ring, slot, off):
            def store(t, acc):
                sl = pl.ds(t - off, TT)
                ring[slot, sl, :] = (ring[slot, sl, :]
                                     + acc.astype(jnp.bfloat16))
            return store

        rr0.wait_recv()
        accum_rows(slot_x(1), 2, 0, nh, add_ring(rsr, 1, 0))
        rr0.wait_send()
        rr1 = rs_hop(rsr, (rsr_send, rsr_recv), 1, right)

        rl0.wait_recv()
        accum_rows(slot_x(1), 2, nh, n_tok, add_ring(rsl, 1, nh))
        rl0.wait_send()
        rl1 = rs_hop(rsl, (rsl_send, rsl_recv), 1, left)

        rr1.wait_recv()
        accum_rows(slot_x(2), 3, 0, nh, add_ring(rsr, 2, 0))
        rr1.wait_send()
        rr2 = rs_hop(rsr, (rsr_send, rsr_recv), 2, right)

        rl1.wait_recv()
        accum_rows(slot_x(0), 1, nh, n_tok, add_ring(rsl, 2, nh))
        rl1.wait_send()
        rl2 = rs_hop(rsl, (rsl_send, rsl_recv), 2, left)

        rr2.wait_recv()
        for t in range(0, nh, TT):
            out_ref[pl.ds(t, TT), :] = (out_ref[pl.ds(t, TT), :]
                                        + rsr[0, pl.ds(t, TT), :]
                                        .astype(jnp.float32))
        rl2.wait_recv()
        for t in range(0, nh, TT):
            out_ref[pl.ds(nh + t, TT), :] = (out_ref[pl.ds(nh + t, TT), :]
                                             + rsl[0, pl.ds(t, TT), :]
                                             .astype(jnp.float32))
        rr2.wait_send()
        rl2.wait_send()

    out = pl.pallas_call(
        body,
        out_shape=jax.ShapeDtypeStruct((n_tok, h), jnp.float32),
        in_specs=[pl.BlockSpec(memory_space=pltpu.VMEM)] * 4,
        out_specs=pl.BlockSpec(memory_space=pltpu.VMEM),
        scratch_shapes=[
            pltpu.VMEM((3, n_tok, d), jnp.bfloat16),
            pltpu.VMEM((N_DEV, n_tok, N_EXP), jnp.bfloat16),
            pltpu.VMEM((3, nh, h), jnp.bfloat16),
            pltpu.VMEM((3, nh, h), jnp.bfloat16),
            pltpu.SemaphoreType.DMA((2,)),
            pltpu.SemaphoreType.DMA((2,)),
            pltpu.SemaphoreType.DMA((2,)),
            pltpu.SemaphoreType.DMA((2,)),
            pltpu.SemaphoreType.DMA((1,)),
            pltpu.SemaphoreType.DMA((1,)),
            pltpu.SemaphoreType.DMA((1,)),
            pltpu.SemaphoreType.DMA((1,)),
            pltpu.SemaphoreType.DMA((3,)),
            pltpu.SemaphoreType.DMA((3,)),
            pltpu.SemaphoreType.DMA((3,)),
            pltpu.SemaphoreType.DMA((3,)),
        ],
        compiler_params=pltpu.CompilerParams(collective_id=0),
    )(
        x.astype(jnp.bfloat16),
        router_W.astype(jnp.bfloat16),
        route_idx,
        expert_W.astype(jnp.bfloat16),
    )
    return out
